# baseline (speedup 1.0000x reference)
import functools

import jax
import jax.numpy as jnp
from jax import lax
from jax.experimental import pallas as pl
from jax.experimental.pallas import tpu as pltpu
from jax.experimental.pallas import tpu_sc as plsc

_mesh = plsc.VectorSubcoreMesh(core_axis_name="c", subcore_axis_name="s")


@functools.partial(
    pl.kernel,
    out_type=jax.ShapeDtypeStruct((32, 16), jnp.float32),
    mesh=_mesh,
    scratch_types=[pltpu.VMEM((16,), jnp.float32)],
)
def _sc_min(mask_hbm, out_hbm, buf):
    wid = lax.axis_index("s") * 2 + lax.axis_index("c")
    pltpu.sync_copy(mask_hbm.at[wid], buf)
    pltpu.sync_copy(buf, out_hbm.at[wid])


def kernel(x, mask):
    m2 = jnp.pad(mask.reshape(32, 4), ((0, 0), (0, 12)))
    return _sc_min(m2)


# speedup vs baseline: 1.7295x; 1.7295x over previous
import jax
import jax.numpy as jnp
from jax.experimental import pallas as pl

R, C = 128, 32768
BR = 64


def _body(x_ref, m_ref, o_ref):
    o_ref[...] = x_ref[...] * m_ref[...]


def kernel(x, mask):
    return pl.pallas_call(
        _body,
        out_shape=jax.ShapeDtypeStruct((R, C), x.dtype),
        grid=(R // BR,),
        in_specs=[
            pl.BlockSpec((BR, C), lambda j: (j, 0)),
            pl.BlockSpec((BR, 1), lambda j: (j, 0)),
        ],
        out_specs=pl.BlockSpec((BR, C), lambda j: (j, 0)),
    )(x, mask[:, None])
